# Initial kernel scaffold; baseline (speedup 1.0000x reference)
import jax
import jax.numpy as jnp
from jax.experimental import pallas as pl


def kernel(x, params, edge_index):
    raise NotImplementedError("write your pallas kernel here")


# speedup vs baseline: 3.5672x; 3.5672x over previous
import jax
import jax.numpy as jnp
from jax import lax
from jax.experimental import pallas as pl
from jax.experimental.pallas import tpu as pltpu
from jax.experimental.pallas import tpu_sc as plsc

NC = 2
NS = 16
CHUNK = 96
DENW = 128


def _sc_mesh():
    return plsc.VectorSubcoreMesh(core_axis_name="c", subcore_axis_name="s",
                                  num_cores=NC, num_subcores=NS)


def _mm(x, b_prev, dnorm, wl, bl, wr, br, halves_in, relu_in):
    dout = wl.shape[1]
    dh = dout // 2
    din = wl.shape[0]
    npad = x.shape[1] if halves_in else x.shape[0]
    rb = min(512, npad)
    grid = (npad // rb,)
    if halves_in:
        x_spec = pl.BlockSpec((2, rb, din // 2), lambda j: (0, j, 0))
    else:
        x_spec = pl.BlockSpec((rb, din), lambda j: (j, 0))
    in_specs = [x_spec]
    args = [x]
    if dnorm is not None:
        in_specs.append(pl.BlockSpec((2, rb, DENW), lambda j: (0, j, 0)))
        args.append(dnorm)
    if b_prev is not None:
        in_specs.append(pl.BlockSpec((1, din), lambda j: (0, 0)))
        args.append(b_prev.reshape(1, din))
    in_specs += [
        pl.BlockSpec((din, dout), lambda j: (0, 0)),
        pl.BlockSpec((1, dout), lambda j: (0, 0)),
        pl.BlockSpec((din, dout), lambda j: (0, 0)),
        pl.BlockSpec((1, dout), lambda j: (0, 0)),
    ]
    args += [wl, bl.reshape(1, dout), wr, br.reshape(1, dout)]

    def body(*refs):
        refs = list(refs)
        x_ref = refs.pop(0)
        dn_ref = refs.pop(0) if dnorm is not None else None
        bp_ref = refs.pop(0) if b_prev is not None else None
        wl_ref, bl_ref, wr_ref, br_ref, xl_ref, xr_ref = refs
        if halves_in:
            h = jnp.concatenate([x_ref[0], x_ref[1]], axis=1)
        else:
            h = x_ref[...]
        if dn_ref is not None:
            d = dn_ref[0, :, 0:1] + dn_ref[1, :, 0:1]
            h = h / jnp.maximum(d, 1e-30)
        if bp_ref is not None:
            h = h + bp_ref[...]
        if relu_in:
            h = jnp.maximum(h, 0.0)
        rl = jnp.dot(h, wl_ref[...], preferred_element_type=jnp.float32,
                     precision=lax.Precision.HIGHEST) + bl_ref[...]
        rr = jnp.dot(h, wr_ref[...], preferred_element_type=jnp.float32,
                     precision=lax.Precision.HIGHEST) + br_ref[...]
        xl_ref[0] = rl[:, :dh]
        xl_ref[1] = rl[:, dh:]
        xr_ref[0] = rr[:, :dh]
        xr_ref[1] = rr[:, dh:]

    return pl.pallas_call(
        body,
        grid=grid,
        in_specs=in_specs,
        out_specs=[pl.BlockSpec((2, rb, dh), lambda j: (0, j, 0)),
                   pl.BlockSpec((2, rb, dh), lambda j: (0, j, 0))],
        out_shape=[jax.ShapeDtypeStruct((2, npad, dh), jnp.float32)] * 2,
    )(*args)


def _final_norm(u4, dp4, bias_scalar):
    npad = u4.shape[1]
    rb = min(512, npad)
    b2 = jnp.full((1, DENW), bias_scalar, jnp.float32)

    def body(u_ref, d_ref, b_ref, o_ref):
        d = d_ref[0, :, 0:1] + d_ref[1, :, 0:1]
        num = u_ref[0] + u_ref[1]
        o_ref[...] = num / jnp.maximum(d, 1e-30) + b_ref[...]

    return pl.pallas_call(
        body,
        grid=(npad // rb,),
        in_specs=[pl.BlockSpec((2, rb, DENW), lambda j: (0, j, 0)),
                  pl.BlockSpec((2, rb, DENW), lambda j: (0, j, 0)),
                  pl.BlockSpec((1, DENW), lambda j: (0, 0))],
        out_specs=pl.BlockSpec((rb, DENW), lambda j: (j, 0)),
        out_shape=jax.ShapeDtypeStruct((npad, DENW), jnp.float32),
    )(u4, dp4, b2)


def _edge_split(E, nway):
    ta = E // nway
    nch = ta // CHUNK
    assert nch * CHUNK == ta, (E, nway, ta)
    return ta, nch


def _phase_a(xlT, xrT, src, dst, att):
    dh = xlT.shape[2]
    E = src.shape[0]
    ta, nch = _edge_split(E, NC * NS)
    datt = att.shape[0]
    nq = dh // 16

    def body(xl_h, xr_h, src_h, dst_h, att_h, ex_h,
             srcv, dstv, rl_lo, rl_hi, rr_lo, rr_hi,
             exbuf, attv, sem):
        cid = lax.axis_index("c")
        sid = lax.axis_index("s")
        wid = sid * NC + cid
        base = wid * ta
        iota = lax.iota(jnp.int32, 16)

        pltpu.sync_copy(att_h, attv)

        def group(g):
            att_lo = [attv[pl.ds(q * 16, 16)] for q in range(nq)]
            att_hi = [attv[pl.ds(dh + q * 16, 16)] for q in range(nq)]
            logits = jnp.zeros((16,), jnp.float32)
            for uu in range(16):
                e = g * 16 + uu
                acc = jnp.zeros((16,), jnp.float32)
                for q in range(nq):
                    sl = pl.ds(q * 16, 16)
                    t = rl_lo[e, sl] + rr_lo[e, sl]
                    t = jnp.maximum(t, 0.2 * t)
                    acc = acc + att_lo[q] * t
                    th = rl_hi[e, sl] + rr_hi[e, sl]
                    th = jnp.maximum(th, 0.2 * th)
                    acc = acc + att_hi[q] * th
                s = acc[0]
                for l in range(1, 16):
                    s = s + acc[l]
                logits = jnp.where(iota == uu, s, logits)
            exbuf[pl.ds(g * 16, 16)] = jnp.exp(logits)

        def chunk(k, _):
            s = base + k * CHUNK
            pltpu.sync_copy(src_h.at[pl.ds(s, CHUNK)], srcv.at[0])
            pltpu.sync_copy(dst_h.at[pl.ds(s, CHUNK)], dstv.at[0])
            cps = [pltpu.async_copy(xl_h.at[0].at[srcv.at[0]], rl_lo, sem),
                   pltpu.async_copy(xl_h.at[1].at[srcv.at[0]], rl_hi, sem),
                   pltpu.async_copy(xr_h.at[0].at[dstv.at[0]], rr_lo, sem),
                   pltpu.async_copy(xr_h.at[1].at[dstv.at[0]], rr_hi, sem)]
            for cp in cps:
                cp.wait()

            def gl(g, _):
                group(g)
                return 0
            lax.fori_loop(0, CHUNK // 16, gl, 0)
            pltpu.sync_copy(exbuf, ex_h.at[pl.ds(s, CHUNK)])
            return 0
        lax.fori_loop(0, nch, chunk, 0)

    kern = pl.kernel(
        body,
        out_type=jax.ShapeDtypeStruct((E,), jnp.float32),
        mesh=_sc_mesh(),
        scratch_types=[
            pltpu.VMEM((2, CHUNK), jnp.int32),
            pltpu.VMEM((2, CHUNK), jnp.int32),
            pltpu.VMEM((CHUNK, dh), jnp.float32),
            pltpu.VMEM((CHUNK, dh), jnp.float32),
            pltpu.VMEM((CHUNK, dh), jnp.float32),
            pltpu.VMEM((CHUNK, dh), jnp.float32),
            pltpu.VMEM((CHUNK,), jnp.float32),
            pltpu.VMEM((max(datt, 16),), jnp.float32),
            pltpu.SemaphoreType.DMA,
        ],
    )
    return kern(xlT, xrT, src, dst, att)


def _phase_b(xlT, src, dst, ex):
    dh = xlT.shape[2]
    npad = xlT.shape[1]
    E = src.shape[0]
    ta, nch_a = _edge_split(E, NC * NS)
    tb, nch_b = _edge_split(E, NS)
    stripe = npad // NS
    zbr = min(16, stripe)
    assert stripe % zbr == 0

    def body(xl_h, src_h, dst_h, ex_h, out_h, dp_h,
             srcv, dstv, rows, exb, zb2, out_sh, sem):
        cid = lax.axis_index("c")
        sid = lax.axis_index("s")
        wid = sid * NC + cid
        zero16 = jnp.zeros((16,), jnp.float32)

        def zero_stripe():
            for t in range(stripe // zbr):
                pltpu.sync_copy(
                    zb2, out_sh.at[pl.ds(sid * stripe + t * zbr, zbr), :])

        for r in range(zbr):
            for q in range(dh // 16):
                zb2[r, pl.ds(q * 16, 16)] = zero16
        zero_stripe()
        plsc.subcore_barrier()

        def den_chunk(k, _):
            s = wid * ta + k * CHUNK
            pltpu.sync_copy(dst_h.at[pl.ds(s, CHUNK)], dstv.at[0])
            pltpu.sync_copy(ex_h.at[pl.ds(s, CHUNK)], exb)

            def bg(g, _):
                exv = exb[pl.ds(g * 16, 16)]
                for uu in range(16):
                    a = exv[uu]
                    e = g * 16 + uu
                    for q in range(dh // 16):
                        rows[e, pl.ds(q * 16, 16)] = jnp.broadcast_to(
                            a, (16,))
                return 0
            lax.fori_loop(0, CHUNK // 16, bg, 0)
            pltpu.sync_copy(rows, out_sh.at[dstv.at[0]], add=True)
            return 0
        lax.fori_loop(0, nch_a, den_chunk, 0)
        plsc.subcore_barrier()
        pltpu.sync_copy(out_sh.at[pl.ds(sid * stripe, stripe), :],
                        dp_h.at[cid, pl.ds(sid * stripe, stripe), :])
        zero_stripe()
        plsc.subcore_barrier()

        def scale_group(g, _):
            exv = exb[pl.ds(g * 16, 16)]
            for uu in range(16):
                a = exv[uu]
                e = g * 16 + uu
                for q in range(dh // 16):
                    sl = pl.ds(q * 16, 16)
                    rows[e, sl] = rows[e, sl] * a
            return 0

        def chunk(k, _):
            s = sid * tb + k * CHUNK
            pltpu.sync_copy(src_h.at[pl.ds(s, CHUNK)], srcv.at[0])
            pltpu.sync_copy(dst_h.at[pl.ds(s, CHUNK)], dstv.at[0])
            pltpu.sync_copy(ex_h.at[pl.ds(s, CHUNK)], exb)
            pltpu.async_copy(xl_h.at[cid].at[srcv.at[0]], rows, sem).wait()
            lax.fori_loop(0, CHUNK // 16, scale_group, 0)
            pltpu.sync_copy(rows, out_sh.at[dstv.at[0]], add=True)
            return 0
        lax.fori_loop(0, nch_b, chunk, 0)
        plsc.subcore_barrier()
        pltpu.sync_copy(out_sh.at[pl.ds(sid * stripe, stripe), :],
                        out_h.at[cid, pl.ds(sid * stripe, stripe), :])

    kern = pl.kernel(
        body,
        out_type=[jax.ShapeDtypeStruct((NC, npad, dh), jnp.float32),
                  jax.ShapeDtypeStruct((NC, npad, DENW), jnp.float32)],
        mesh=_sc_mesh(),
        scratch_types=[
            pltpu.VMEM((2, CHUNK), jnp.int32),
            pltpu.VMEM((2, CHUNK), jnp.int32),
            pltpu.VMEM((CHUNK, dh), jnp.float32),
            pltpu.VMEM((CHUNK,), jnp.float32),
            pltpu.VMEM((min(16, npad // NS), dh), jnp.float32),
            pltpu.VMEM_SHARED((npad, dh), jnp.float32),
            pltpu.SemaphoreType.DMA,
        ],
    )
    return kern(xlT, src, dst, ex)


def _phase_l4(xl4, xr4, src, dst, att4):
    npad = xl4.shape[0]
    E = src.shape[0]
    ta, nch = _edge_split(E, NC * NS)
    stripe = npad // NS
    zbr = min(16, stripe)

    def body(xl_h, xr_h, src_h, dst_h, att_h, u_h, dp_h,
             srcv, dstv, xsg, xrg, rows, attv, zb, acc_sh, sem):
        cid = lax.axis_index("c")
        sid = lax.axis_index("s")
        wid = sid * NC + cid
        base = wid * ta
        zero16 = jnp.zeros((16,), jnp.float32)

        def zero_stripe():
            for t in range(stripe // zbr):
                pltpu.sync_copy(
                    zb, acc_sh.at[pl.ds(sid * stripe + t * zbr, zbr), :])

        for r in range(zbr):
            for q in range(DENW // 16):
                zb[r, pl.ds(q * 16, 16)] = zero16
        zero_stripe()
        pltpu.sync_copy(att_h, attv)
        plsc.subcore_barrier()
        a0 = attv[pl.ds(0, 16)][0]

        def pass_edges(numerator):
            def chunk(k, _):
                s = base + k * CHUNK
                pltpu.sync_copy(src_h.at[pl.ds(s, CHUNK)], srcv.at[0])
                pltpu.sync_copy(dst_h.at[pl.ds(s, CHUNK)], dstv.at[0])
                cps = [pltpu.async_copy(xl_h.at[srcv.at[0]], xsg, sem),
                       pltpu.async_copy(xr_h.at[dstv.at[0]], xrg, sem)]
                for cp in cps:
                    cp.wait()

                def bg(g, _):
                    sl = pl.ds(g * 16, 16)
                    l = xsg[sl]
                    t = l + xrg[sl]
                    t = jnp.maximum(t, 0.2 * t)
                    v = jnp.exp(a0 * t)
                    if numerator:
                        v = v * l
                    for uu in range(16):
                        e = g * 16 + uu
                        for q in range(DENW // 16):
                            rows[e, pl.ds(q * 16, 16)] = jnp.broadcast_to(
                                v[uu], (16,))
                    return 0
                lax.fori_loop(0, CHUNK // 16, bg, 0)
                pltpu.sync_copy(rows, acc_sh.at[dstv.at[0]], add=True)
                return 0
            lax.fori_loop(0, nch, chunk, 0)

        pass_edges(False)
        plsc.subcore_barrier()
        off = pl.ds(sid * stripe, stripe)
        pltpu.sync_copy(acc_sh.at[off, :], dp_h.at[cid, off, :])
        zero_stripe()
        plsc.subcore_barrier()
        pass_edges(True)
        plsc.subcore_barrier()
        pltpu.sync_copy(acc_sh.at[off, :], u_h.at[cid, off, :])

    kern = pl.kernel(
        body,
        out_type=[jax.ShapeDtypeStruct((NC, npad, DENW), jnp.float32),
                  jax.ShapeDtypeStruct((NC, npad, DENW), jnp.float32)],
        mesh=_sc_mesh(),
        scratch_types=[
            pltpu.VMEM((2, CHUNK), jnp.int32),
            pltpu.VMEM((2, CHUNK), jnp.int32),
            pltpu.VMEM((CHUNK,), jnp.float32),
            pltpu.VMEM((CHUNK,), jnp.float32),
            pltpu.VMEM((CHUNK, DENW), jnp.float32),
            pltpu.VMEM((16,), jnp.float32),
            pltpu.VMEM((min(16, npad // NS), DENW), jnp.float32),
            pltpu.VMEM_SHARED((npad, DENW), jnp.float32),
            pltpu.SemaphoreType.DMA,
        ],
    )
    return kern(xl4, xr4, src, dst, att4)


def _gat_layer(h_or_x, b_prev, dnorm, p, src, dst, halves_in, relu_in):
    xlT, xrT = _mm(h_or_x, b_prev, dnorm, p["Wl"], p["bl"], p["Wr"], p["br"],
                   halves_in, relu_in)
    ex = _phase_a(xlT, xrT, src, dst, p["att"])
    u, dp = _phase_b(xlT, src, dst, ex)
    return u, dp


def kernel(x, params, edge_index):
    n, _ = x.shape
    npad = ((n + 16 + 255) // 256) * 256
    xp = jnp.pad(x, ((0, npad - n), (0, 0)))
    e = edge_index.shape[1]
    epad = -(-e // (CHUNK * NC * NS)) * (CHUNK * NC * NS)
    dump = npad - 16
    src = jnp.pad(edge_index[0], (0, epad - e))
    dst = jnp.pad(edge_index[1], (0, epad - e), constant_values=dump)
    p1, p2, p3, p4 = (params["l1"], params["l2"], params["l3"], params["l4"])

    u, dp = _gat_layer(xp, None, None, p1, src, dst,
                       halves_in=False, relu_in=False)
    u, dp = _gat_layer(u, p1["bias"], dp, p2, src, dst,
                       halves_in=True, relu_in=True)
    u, dp = _gat_layer(u, p2["bias"], dp, p3, src, dst,
                       halves_in=True, relu_in=True)

    pad = 128
    wl4 = jnp.pad(p4["Wl"], ((0, 0), (0, pad - 1)))
    wr4 = jnp.pad(p4["Wr"], ((0, 0), (0, pad - 1)))
    bl4 = jnp.pad(p4["bl"], (0, pad - 1))
    br4 = jnp.pad(p4["br"], (0, pad - 1))
    xlT4, xrT4 = _mm(u, p3["bias"], dp, wl4, bl4, wr4, br4,
                     halves_in=True, relu_in=True)
    xl4 = xlT4[0, :, 0]
    xr4 = xrT4[0, :, 0]
    att4 = jnp.pad(p4["att"], (0, 15))
    u4, dp4 = _phase_l4(xl4, xr4, src, dst, att4)
    o = _final_norm(u4, dp4, params["l4"]["bias"][0])
    return o[:n, 0]


# speedup vs baseline: 4.9437x; 1.3859x over previous
import jax
import jax.numpy as jnp
from jax import lax
from jax.experimental import pallas as pl
from jax.experimental.pallas import tpu as pltpu
from jax.experimental.pallas import tpu_sc as plsc

NC = 2
NS = 16
CHUNK = 96
DENW = 128


def _sc_mesh():
    return plsc.VectorSubcoreMesh(core_axis_name="c", subcore_axis_name="s",
                                  num_cores=NC, num_subcores=NS)


def _mm(x, b_prev, dnorm, wl, bl, wr, br, halves_in, relu_in):
    dout = wl.shape[1]
    dh = dout // 2
    din = wl.shape[0]
    npad = x.shape[1] if halves_in else x.shape[0]
    rb = min(512, npad)
    grid = (npad // rb,)
    if halves_in:
        x_spec = pl.BlockSpec((2, rb, din // 2), lambda j: (0, j, 0))
    else:
        x_spec = pl.BlockSpec((rb, din), lambda j: (j, 0))
    in_specs = [x_spec]
    args = [x]
    if dnorm is not None:
        in_specs.append(pl.BlockSpec((2, rb, DENW), lambda j: (0, j, 0)))
        args.append(dnorm)
    if b_prev is not None:
        in_specs.append(pl.BlockSpec((1, din), lambda j: (0, 0)))
        args.append(b_prev.reshape(1, din))
    in_specs += [
        pl.BlockSpec((din, dout), lambda j: (0, 0)),
        pl.BlockSpec((1, dout), lambda j: (0, 0)),
        pl.BlockSpec((din, dout), lambda j: (0, 0)),
        pl.BlockSpec((1, dout), lambda j: (0, 0)),
    ]
    args += [wl, bl.reshape(1, dout), wr, br.reshape(1, dout)]

    def body(*refs):
        refs = list(refs)
        x_ref = refs.pop(0)
        dn_ref = refs.pop(0) if dnorm is not None else None
        bp_ref = refs.pop(0) if b_prev is not None else None
        wl_ref, bl_ref, wr_ref, br_ref, xl_ref, xr_ref = refs
        if halves_in:
            h = jnp.concatenate([x_ref[0], x_ref[1]], axis=1)
        else:
            h = x_ref[...]
        if dn_ref is not None:
            d = dn_ref[0, :, 0:1] + dn_ref[1, :, 0:1]
            h = h / jnp.maximum(d, 1e-30)
        if bp_ref is not None:
            h = h + bp_ref[...]
        if relu_in:
            h = jnp.maximum(h, 0.0)
        rl = jnp.dot(h, wl_ref[...], preferred_element_type=jnp.float32,
                     precision=lax.Precision.HIGHEST) + bl_ref[...]
        rr = jnp.dot(h, wr_ref[...], preferred_element_type=jnp.float32,
                     precision=lax.Precision.HIGHEST) + br_ref[...]
        xl_ref[0] = rl[:, :dh]
        xl_ref[1] = rl[:, dh:]
        xr_ref[0] = rr[:, :dh]
        xr_ref[1] = rr[:, dh:]

    return pl.pallas_call(
        body,
        grid=grid,
        in_specs=in_specs,
        out_specs=[pl.BlockSpec((2, rb, dh), lambda j: (0, j, 0)),
                   pl.BlockSpec((2, rb, dh), lambda j: (0, j, 0))],
        out_shape=[jax.ShapeDtypeStruct((2, npad, dh), jnp.float32)] * 2,
    )(*args)


def _final_norm(u4, dp4, bias_scalar):
    npad = u4.shape[1]
    rb = min(512, npad)
    b2 = jnp.full((1, DENW), bias_scalar, jnp.float32)

    def body(u_ref, d_ref, b_ref, o_ref):
        d = d_ref[0, :, 0:1] + d_ref[1, :, 0:1]
        num = u_ref[0] + u_ref[1]
        o_ref[...] = num / jnp.maximum(d, 1e-30) + b_ref[...]

    return pl.pallas_call(
        body,
        grid=(npad // rb,),
        in_specs=[pl.BlockSpec((2, rb, DENW), lambda j: (0, j, 0)),
                  pl.BlockSpec((2, rb, DENW), lambda j: (0, j, 0)),
                  pl.BlockSpec((1, DENW), lambda j: (0, 0))],
        out_specs=pl.BlockSpec((rb, DENW), lambda j: (j, 0)),
        out_shape=jax.ShapeDtypeStruct((npad, DENW), jnp.float32),
    )(u4, dp4, b2)


def _edge_split(E, nway):
    ta = E // nway
    nch = ta // CHUNK
    assert nch * CHUNK == ta, (E, nway, ta)
    return ta, nch


def _phase_a(xlT, xrT, src, dst, att):
    dh = xlT.shape[2]
    E = src.shape[0]
    ta, nch = _edge_split(E, NC * NS)
    datt = att.shape[0]
    nq = dh // 16

    def body(xl_h, xr_h, src_h, dst_h, att_h, ex_h,
             srcv, dstv, rl_lo, rl_hi, rr_lo, rr_hi,
             exbuf, attv, sem, sem2):
        cid = lax.axis_index("c")
        sid = lax.axis_index("s")
        wid = sid * NC + cid
        base = wid * ta
        iota = lax.iota(jnp.int32, 16)

        pltpu.sync_copy(att_h, attv)

        def group(j, g):
            att_lo = [attv[pl.ds(q * 16, 16)] for q in range(nq)]
            att_hi = [attv[pl.ds(dh + q * 16, 16)] for q in range(nq)]
            logits = jnp.zeros((16,), jnp.float32)
            for uu in range(16):
                e = j * CHUNK + g * 16 + uu
                acc = jnp.zeros((16,), jnp.float32)
                for q in range(nq):
                    sl = pl.ds(q * 16, 16)
                    t = rl_lo[e, sl] + rr_lo[e, sl]
                    t = jnp.maximum(t, 0.2 * t)
                    acc = acc + att_lo[q] * t
                    th = rl_hi[e, sl] + rr_hi[e, sl]
                    th = jnp.maximum(th, 0.2 * th)
                    acc = acc + att_hi[q] * th
                s = acc[0]
                for l in range(1, 16):
                    s = s + acc[l]
                logits = jnp.where(iota == uu, s, logits)
            exbuf[pl.ds(j * CHUNK + g * 16, 16)] = jnp.exp(logits)

        def fire(k, j, sm):
            s = base + k * CHUNK
            pltpu.sync_copy(src_h.at[pl.ds(s, CHUNK)], srcv.at[j])
            pltpu.sync_copy(dst_h.at[pl.ds(s, CHUNK)], dstv.at[j])
            bsl = pl.ds(j * CHUNK, CHUNK)
            pltpu.async_copy(xl_h.at[0].at[srcv.at[j]], rl_lo.at[bsl], sm)
            pltpu.async_copy(xl_h.at[1].at[srcv.at[j]], rl_hi.at[bsl], sm)
            pltpu.async_copy(xr_h.at[0].at[dstv.at[j]], rr_lo.at[bsl], sm)
            pltpu.async_copy(xr_h.at[1].at[dstv.at[j]], rr_hi.at[bsl], sm)

        def drain(j, sm):
            bsl = pl.ds(j * CHUNK, CHUNK)
            dummy = xl_h.at[0].at[pl.ds(0, CHUNK)]
            for buf in (rl_lo, rl_hi, rr_lo, rr_hi):
                pltpu.make_async_copy(dummy, buf.at[bsl], sm).wait()

        def comp_store(k, j):
            s = base + k * CHUNK

            def gl(g, _):
                group(j, g)
                return 0
            lax.fori_loop(0, CHUNK // 16, gl, 0)
            pltpu.sync_copy(exbuf.at[pl.ds(j * CHUNK, CHUNK)],
                            ex_h.at[pl.ds(s, CHUNK)])

        fire(0, 0, sem)

        def pair(k2, _):
            k = k2 * 2
            fire(k + 1, 1, sem2)
            drain(0, sem)
            comp_store(k, 0)

            @pl.when(k + 2 < nch)
            def _():
                fire(k + 2, 0, sem)
            drain(1, sem2)
            comp_store(k + 1, 1)
            return 0
        lax.fori_loop(0, nch // 2, pair, 0)
        if nch % 2:
            drain(0, sem)
            comp_store(nch - 1, 0)

    kern = pl.kernel(
        body,
        out_type=jax.ShapeDtypeStruct((E,), jnp.float32),
        mesh=_sc_mesh(),
        scratch_types=[
            pltpu.VMEM((2, CHUNK), jnp.int32),
            pltpu.VMEM((2, CHUNK), jnp.int32),
            pltpu.VMEM((2 * CHUNK, dh), jnp.float32),
            pltpu.VMEM((2 * CHUNK, dh), jnp.float32),
            pltpu.VMEM((2 * CHUNK, dh), jnp.float32),
            pltpu.VMEM((2 * CHUNK, dh), jnp.float32),
            pltpu.VMEM((2 * CHUNK,), jnp.float32),
            pltpu.VMEM((max(datt, 16),), jnp.float32),
            pltpu.SemaphoreType.DMA,
            pltpu.SemaphoreType.DMA,
        ],
    )
    return kern(xlT, xrT, src, dst, att)


def _phase_b(xlT, src, dst, ex):
    dh = xlT.shape[2]
    npad = xlT.shape[1]
    E = src.shape[0]
    ta, nch_a = _edge_split(E, NC * NS)
    tb, nch_b = _edge_split(E, NS)
    stripe = npad // NS
    zbr = min(16, stripe)
    assert stripe % zbr == 0

    def body(xl_h, src_h, dst_h, ex_h, out_h, dp_h,
             srcv, dstv, rows, exb, zb2, out_sh, sem, sem2):
        cid = lax.axis_index("c")
        sid = lax.axis_index("s")
        wid = sid * NC + cid
        zero16 = jnp.zeros((16,), jnp.float32)

        def zero_stripe():
            for t in range(stripe // zbr):
                pltpu.sync_copy(
                    zb2, out_sh.at[pl.ds(sid * stripe + t * zbr, zbr), :])

        for r in range(zbr):
            for q in range(dh // 16):
                zb2[r, pl.ds(q * 16, 16)] = zero16
        zero_stripe()
        plsc.subcore_barrier()

        def den_chunk(k, _):
            s = wid * ta + k * CHUNK
            pltpu.sync_copy(dst_h.at[pl.ds(s, CHUNK)], dstv.at[0])
            pltpu.sync_copy(ex_h.at[pl.ds(s, CHUNK)], exb.at[pl.ds(0, CHUNK)])

            def bg(g, _):
                exv = exb[pl.ds(g * 16, 16)]
                for uu in range(16):
                    a = exv[uu]
                    e = g * 16 + uu
                    for q in range(dh // 16):
                        rows[e, pl.ds(q * 16, 16)] = jnp.broadcast_to(
                            a, (16,))
                return 0
            lax.fori_loop(0, CHUNK // 16, bg, 0)
            pltpu.sync_copy(rows.at[pl.ds(0, CHUNK)],
                            out_sh.at[dstv.at[0]], add=True)
            return 0
        lax.fori_loop(0, nch_a, den_chunk, 0)
        plsc.subcore_barrier()
        pltpu.sync_copy(out_sh.at[pl.ds(sid * stripe, stripe), :],
                        dp_h.at[cid, pl.ds(sid * stripe, stripe), :])
        zero_stripe()
        plsc.subcore_barrier()

        def scale_group2(j, g):
            exv = exb[pl.ds(j * CHUNK + g * 16, 16)]
            for uu in range(16):
                a = exv[uu]
                e = j * CHUNK + g * 16 + uu
                for q in range(dh // 16):
                    sl = pl.ds(q * 16, 16)
                    rows[e, sl] = rows[e, sl] * a

        def fire2(k, j, sm):
            s = sid * tb + k * CHUNK
            pltpu.sync_copy(src_h.at[pl.ds(s, CHUNK)], srcv.at[j])
            pltpu.sync_copy(dst_h.at[pl.ds(s, CHUNK)], dstv.at[j])
            pltpu.sync_copy(ex_h.at[pl.ds(s, CHUNK)],
                            exb.at[pl.ds(j * CHUNK, CHUNK)])
            pltpu.async_copy(xl_h.at[cid].at[srcv.at[j]],
                             rows.at[pl.ds(j * CHUNK, CHUNK)], sm)

        def finish2(j, sm):
            dummy = xl_h.at[0].at[pl.ds(0, CHUNK)]
            pltpu.make_async_copy(
                dummy, rows.at[pl.ds(j * CHUNK, CHUNK)], sm).wait()

            def sg(g, _):
                scale_group2(j, g)
                return 0
            lax.fori_loop(0, CHUNK // 16, sg, 0)
            pltpu.sync_copy(rows.at[pl.ds(j * CHUNK, CHUNK)],
                            out_sh.at[dstv.at[j]], add=True)

        fire2(0, 0, sem)

        def pair2(k2, _):
            k = k2 * 2
            fire2(k + 1, 1, sem2)
            finish2(0, sem)

            @pl.when(k + 2 < nch_b)
            def _():
                fire2(k + 2, 0, sem)
            finish2(1, sem2)
            return 0
        lax.fori_loop(0, nch_b // 2, pair2, 0)
        if nch_b % 2:
            finish2(0, sem)
        plsc.subcore_barrier()
        pltpu.sync_copy(out_sh.at[pl.ds(sid * stripe, stripe), :],
                        out_h.at[cid, pl.ds(sid * stripe, stripe), :])

    kern = pl.kernel(
        body,
        out_type=[jax.ShapeDtypeStruct((NC, npad, dh), jnp.float32),
                  jax.ShapeDtypeStruct((NC, npad, DENW), jnp.float32)],
        mesh=_sc_mesh(),
        scratch_types=[
            pltpu.VMEM((2, CHUNK), jnp.int32),
            pltpu.VMEM((2, CHUNK), jnp.int32),
            pltpu.VMEM((2 * CHUNK, dh), jnp.float32),
            pltpu.VMEM((2 * CHUNK,), jnp.float32),
            pltpu.VMEM((min(16, npad // NS), dh), jnp.float32),
            pltpu.VMEM_SHARED((npad, dh), jnp.float32),
            pltpu.SemaphoreType.DMA,
            pltpu.SemaphoreType.DMA,
        ],
    )
    return kern(xlT, src, dst, ex)


def _phase_l4(xl4, xr4, src, dst, att4):
    npad = xl4.shape[0]
    E = src.shape[0]
    ta, nch = _edge_split(E, NC * NS)
    stripe = npad // NS
    zbr = min(16, stripe)

    def body(xl_h, xr_h, src_h, dst_h, att_h, u_h, dp_h,
             srcv, dstv, xsg, xrg, rows, attv, zb, acc_sh, sem):
        cid = lax.axis_index("c")
        sid = lax.axis_index("s")
        wid = sid * NC + cid
        base = wid * ta
        zero16 = jnp.zeros((16,), jnp.float32)

        def zero_stripe():
            for t in range(stripe // zbr):
                pltpu.sync_copy(
                    zb, acc_sh.at[pl.ds(sid * stripe + t * zbr, zbr), :])

        for r in range(zbr):
            for q in range(DENW // 16):
                zb[r, pl.ds(q * 16, 16)] = zero16
        zero_stripe()
        pltpu.sync_copy(att_h, attv)
        plsc.subcore_barrier()
        a0 = attv[pl.ds(0, 16)][0]

        def pass_edges(numerator):
            def chunk(k, _):
                s = base + k * CHUNK
                pltpu.sync_copy(src_h.at[pl.ds(s, CHUNK)], srcv.at[0])
                pltpu.sync_copy(dst_h.at[pl.ds(s, CHUNK)], dstv.at[0])
                cps = [pltpu.async_copy(xl_h.at[srcv.at[0]], xsg, sem),
                       pltpu.async_copy(xr_h.at[dstv.at[0]], xrg, sem)]
                for cp in cps:
                    cp.wait()

                def bg(g, _):
                    sl = pl.ds(g * 16, 16)
                    l = xsg[sl]
                    t = l + xrg[sl]
                    t = jnp.maximum(t, 0.2 * t)
                    v = jnp.exp(a0 * t)
                    if numerator:
                        v = v * l
                    for uu in range(16):
                        e = g * 16 + uu
                        for q in range(DENW // 16):
                            rows[e, pl.ds(q * 16, 16)] = jnp.broadcast_to(
                                v[uu], (16,))
                    return 0
                lax.fori_loop(0, CHUNK // 16, bg, 0)
                pltpu.sync_copy(rows, acc_sh.at[dstv.at[0]], add=True)
                return 0
            lax.fori_loop(0, nch, chunk, 0)

        pass_edges(False)
        plsc.subcore_barrier()
        off = pl.ds(sid * stripe, stripe)
        pltpu.sync_copy(acc_sh.at[off, :], dp_h.at[cid, off, :])
        zero_stripe()
        plsc.subcore_barrier()
        pass_edges(True)
        plsc.subcore_barrier()
        pltpu.sync_copy(acc_sh.at[off, :], u_h.at[cid, off, :])

    kern = pl.kernel(
        body,
        out_type=[jax.ShapeDtypeStruct((NC, npad, DENW), jnp.float32),
                  jax.ShapeDtypeStruct((NC, npad, DENW), jnp.float32)],
        mesh=_sc_mesh(),
        scratch_types=[
            pltpu.VMEM((2, CHUNK), jnp.int32),
            pltpu.VMEM((2, CHUNK), jnp.int32),
            pltpu.VMEM((CHUNK,), jnp.float32),
            pltpu.VMEM((CHUNK,), jnp.float32),
            pltpu.VMEM((CHUNK, DENW), jnp.float32),
            pltpu.VMEM((16,), jnp.float32),
            pltpu.VMEM((min(16, npad // NS), DENW), jnp.float32),
            pltpu.VMEM_SHARED((npad, DENW), jnp.float32),
            pltpu.SemaphoreType.DMA,
        ],
    )
    return kern(xl4, xr4, src, dst, att4)


def _gat_layer(h_or_x, b_prev, dnorm, p, src, dst, halves_in, relu_in):
    xlT, xrT = _mm(h_or_x, b_prev, dnorm, p["Wl"], p["bl"], p["Wr"], p["br"],
                   halves_in, relu_in)
    ex = _phase_a(xlT, xrT, src, dst, p["att"])
    u, dp = _phase_b(xlT, src, dst, ex)
    return u, dp


def kernel(x, params, edge_index):
    n, _ = x.shape
    npad = ((n + 16 + 255) // 256) * 256
    xp = jnp.pad(x, ((0, npad - n), (0, 0)))
    e = edge_index.shape[1]
    epad = -(-e // (CHUNK * NC * NS)) * (CHUNK * NC * NS)
    dump = npad - 16
    src = jnp.pad(edge_index[0], (0, epad - e))
    dst = jnp.pad(edge_index[1], (0, epad - e), constant_values=dump)
    p1, p2, p3, p4 = (params["l1"], params["l2"], params["l3"], params["l4"])

    u, dp = _gat_layer(xp, None, None, p1, src, dst,
                       halves_in=False, relu_in=False)
    u, dp = _gat_layer(u, p1["bias"], dp, p2, src, dst,
                       halves_in=True, relu_in=True)
    u, dp = _gat_layer(u, p2["bias"], dp, p3, src, dst,
                       halves_in=True, relu_in=True)

    pad = 128
    wl4 = jnp.pad(p4["Wl"], ((0, 0), (0, pad - 1)))
    wr4 = jnp.pad(p4["Wr"], ((0, 0), (0, pad - 1)))
    bl4 = jnp.pad(p4["bl"], (0, pad - 1))
    br4 = jnp.pad(p4["br"], (0, pad - 1))
    xlT4, xrT4 = _mm(u, p3["bias"], dp, wl4, bl4, wr4, br4,
                     halves_in=True, relu_in=True)
    xl4 = xlT4[0, :, 0]
    xr4 = xrT4[0, :, 0]
    att4 = jnp.pad(p4["att"], (0, 15))
    u4, dp4 = _phase_l4(xl4, xr4, src, dst, att4)
    o = _final_norm(u4, dp4, params["l4"]["bias"][0])
    return o[:n, 0]


# speedup vs baseline: 5.1628x; 1.0443x over previous
import jax
import jax.numpy as jnp
from jax import lax
from jax.experimental import pallas as pl
from jax.experimental.pallas import tpu as pltpu
from jax.experimental.pallas import tpu_sc as plsc

NC = 2
NS = 16
CHUNK = 96
DENW = 128


def _sc_mesh():
    return plsc.VectorSubcoreMesh(core_axis_name="c", subcore_axis_name="s",
                                  num_cores=NC, num_subcores=NS)


def _mm(x, b_prev, dnorm, wl, bl, wr, br, halves_in, relu_in):
    dout = wl.shape[1]
    dh = dout // 2
    din = wl.shape[0]
    npad = x.shape[1] if halves_in else x.shape[0]
    rb = min(512, npad)
    grid = (npad // rb,)
    if halves_in:
        x_spec = pl.BlockSpec((2, rb, din // 2), lambda j: (0, j, 0))
    else:
        x_spec = pl.BlockSpec((rb, din), lambda j: (j, 0))
    in_specs = [x_spec]
    args = [x]
    if dnorm is not None:
        in_specs.append(pl.BlockSpec((2, rb, DENW), lambda j: (0, j, 0)))
        args.append(dnorm)
    if b_prev is not None:
        in_specs.append(pl.BlockSpec((1, din), lambda j: (0, 0)))
        args.append(b_prev.reshape(1, din))
    in_specs += [
        pl.BlockSpec((din, dout), lambda j: (0, 0)),
        pl.BlockSpec((1, dout), lambda j: (0, 0)),
        pl.BlockSpec((din, dout), lambda j: (0, 0)),
        pl.BlockSpec((1, dout), lambda j: (0, 0)),
    ]
    args += [wl, bl.reshape(1, dout), wr, br.reshape(1, dout)]

    def body(*refs):
        refs = list(refs)
        x_ref = refs.pop(0)
        dn_ref = refs.pop(0) if dnorm is not None else None
        bp_ref = refs.pop(0) if b_prev is not None else None
        wl_ref, bl_ref, wr_ref, br_ref, xl_ref, xr_ref = refs
        if halves_in:
            h = jnp.concatenate([x_ref[0], x_ref[1]], axis=1)
        else:
            h = x_ref[...]
        if dn_ref is not None:
            d = dn_ref[0, :, 0:1] + dn_ref[1, :, 0:1]
            h = h / jnp.maximum(d, 1e-30)
        if bp_ref is not None:
            h = h + bp_ref[...]
        if relu_in:
            h = jnp.maximum(h, 0.0)
        rl = jnp.dot(h, wl_ref[...], preferred_element_type=jnp.float32,
                     precision=lax.Precision.HIGHEST) + bl_ref[...]
        rr = jnp.dot(h, wr_ref[...], preferred_element_type=jnp.float32,
                     precision=lax.Precision.HIGHEST) + br_ref[...]
        xl_ref[0] = rl[:, :dh]
        xl_ref[1] = rl[:, dh:]
        xr_ref[0] = rr[:, :dh]
        xr_ref[1] = rr[:, dh:]

    return pl.pallas_call(
        body,
        grid=grid,
        in_specs=in_specs,
        out_specs=[pl.BlockSpec((2, rb, dh), lambda j: (0, j, 0)),
                   pl.BlockSpec((2, rb, dh), lambda j: (0, j, 0))],
        out_shape=[jax.ShapeDtypeStruct((2, npad, dh), jnp.float32)] * 2,
    )(*args)


def _final_norm(u4, dp4, bias_scalar):
    npad = u4.shape[1]
    rb = min(512, npad)
    b2 = jnp.full((1, DENW), bias_scalar, jnp.float32)

    def body(u_ref, d_ref, b_ref, o_ref):
        d = d_ref[0, :, 0:1] + d_ref[1, :, 0:1]
        num = u_ref[0] + u_ref[1]
        o_ref[...] = num / jnp.maximum(d, 1e-30) + b_ref[...]

    return pl.pallas_call(
        body,
        grid=(npad // rb,),
        in_specs=[pl.BlockSpec((2, rb, DENW), lambda j: (0, j, 0)),
                  pl.BlockSpec((2, rb, DENW), lambda j: (0, j, 0)),
                  pl.BlockSpec((1, DENW), lambda j: (0, 0))],
        out_specs=pl.BlockSpec((rb, DENW), lambda j: (j, 0)),
        out_shape=jax.ShapeDtypeStruct((npad, DENW), jnp.float32),
    )(u4, dp4, b2)


def _edge_split(E, nway):
    ta = E // nway
    nch = ta // CHUNK
    assert nch * CHUNK == ta, (E, nway, ta)
    return ta, nch


def _phase_a(xlT, xrT, src, dst, att):
    dh = xlT.shape[2]
    E = src.shape[0]
    ta, nch = _edge_split(E, NC * NS)
    datt = att.shape[0]
    nq = dh // 16

    def body(xl_h, xr_h, src_h, dst_h, att_h, ex_h,
             srcv, dstv, rl_lo, rl_hi, rr_lo, rr_hi,
             exbuf, attv, sem, sem2):
        cid = lax.axis_index("c")
        sid = lax.axis_index("s")
        wid = sid * NC + cid
        base = wid * ta
        iota = lax.iota(jnp.int32, 16)

        pltpu.sync_copy(att_h, attv)

        def group(j, g):
            att_lo = [attv[pl.ds(q * 16, 16)] for q in range(nq)]
            att_hi = [attv[pl.ds(dh + q * 16, 16)] for q in range(nq)]
            logits = jnp.zeros((16,), jnp.float32)
            for uu in range(16):
                e = j * CHUNK + g * 16 + uu
                acc = jnp.zeros((16,), jnp.float32)
                for q in range(nq):
                    sl = pl.ds(q * 16, 16)
                    t = rl_lo[e, sl] + rr_lo[e, sl]
                    t = jnp.maximum(t, 0.2 * t)
                    acc = acc + att_lo[q] * t
                    th = rl_hi[e, sl] + rr_hi[e, sl]
                    th = jnp.maximum(th, 0.2 * th)
                    acc = acc + att_hi[q] * th
                s = acc[0]
                for l in range(1, 16):
                    s = s + acc[l]
                logits = jnp.where(iota == uu, s, logits)
            exbuf[pl.ds(j * CHUNK + g * 16, 16)] = jnp.exp(logits)

        def fire(k, j, sm):
            s = base + k * CHUNK
            pltpu.sync_copy(src_h.at[pl.ds(s, CHUNK)], srcv.at[j])
            pltpu.sync_copy(dst_h.at[pl.ds(s, CHUNK)], dstv.at[j])
            bsl = pl.ds(j * CHUNK, CHUNK)
            pltpu.async_copy(xl_h.at[0].at[srcv.at[j]], rl_lo.at[bsl], sm)
            pltpu.async_copy(xl_h.at[1].at[srcv.at[j]], rl_hi.at[bsl], sm)
            pltpu.async_copy(xr_h.at[0].at[dstv.at[j]], rr_lo.at[bsl], sm)
            pltpu.async_copy(xr_h.at[1].at[dstv.at[j]], rr_hi.at[bsl], sm)

        def drain(j, sm):
            bsl = pl.ds(j * CHUNK, CHUNK)
            dummy = xl_h.at[0].at[pl.ds(0, CHUNK)]
            for buf in (rl_lo, rl_hi, rr_lo, rr_hi):
                pltpu.make_async_copy(dummy, buf.at[bsl], sm).wait()

        def comp_store(k, j):
            s = base + k * CHUNK

            def gl(g, _):
                group(j, g)
                return 0
            lax.fori_loop(0, CHUNK // 16, gl, 0)
            pltpu.sync_copy(exbuf.at[pl.ds(j * CHUNK, CHUNK)],
                            ex_h.at[pl.ds(s, CHUNK)])

        fire(0, 0, sem)

        def pair(k2, _):
            k = k2 * 2
            fire(k + 1, 1, sem2)
            drain(0, sem)
            comp_store(k, 0)

            @pl.when(k + 2 < nch)
            def _():
                fire(k + 2, 0, sem)
            drain(1, sem2)
            comp_store(k + 1, 1)
            return 0
        lax.fori_loop(0, nch // 2, pair, 0)
        if nch % 2:
            drain(0, sem)
            comp_store(nch - 1, 0)

    kern = pl.kernel(
        body,
        out_type=jax.ShapeDtypeStruct((E,), jnp.float32),
        mesh=_sc_mesh(),
        scratch_types=[
            pltpu.VMEM((2, CHUNK), jnp.int32),
            pltpu.VMEM((2, CHUNK), jnp.int32),
            pltpu.VMEM((2 * CHUNK, dh), jnp.float32),
            pltpu.VMEM((2 * CHUNK, dh), jnp.float32),
            pltpu.VMEM((2 * CHUNK, dh), jnp.float32),
            pltpu.VMEM((2 * CHUNK, dh), jnp.float32),
            pltpu.VMEM((2 * CHUNK,), jnp.float32),
            pltpu.VMEM((max(datt, 16),), jnp.float32),
            pltpu.SemaphoreType.DMA,
            pltpu.SemaphoreType.DMA,
        ],
    )
    return kern(xlT, xrT, src, dst, att)


def _phase_b(xlT, src, dst, ex):
    dh = xlT.shape[2]
    npad = xlT.shape[1]
    E = src.shape[0]
    ta, nch_a = _edge_split(E, NC * NS)
    tb, nch_b = _edge_split(E, NS)
    stripe = npad // NS
    zbr = min(16, stripe)
    assert stripe % zbr == 0

    def body(xl_h, src_h, dst_h, ex_h, out_h, dp_h,
             srcv, dstv, rows, exb, zb2, out_sh, sem, sem2):
        cid = lax.axis_index("c")
        sid = lax.axis_index("s")
        wid = sid * NC + cid
        zero16 = jnp.zeros((16,), jnp.float32)

        def zero_stripe():
            for t in range(stripe // zbr):
                pltpu.sync_copy(
                    zb2, out_sh.at[pl.ds(sid * stripe + t * zbr, zbr), :])

        for r in range(zbr):
            for q in range(dh // 16):
                zb2[r, pl.ds(q * 16, 16)] = zero16
        zero_stripe()
        plsc.subcore_barrier()

        def build1(j):
            def bg(g, _):
                exv = exb[pl.ds(j * CHUNK + g * 16, 16)]
                for uu in range(16):
                    a = exv[uu]
                    e = j * CHUNK + g * 16 + uu
                    for q in range(dh // 16):
                        rows[e, pl.ds(q * 16, 16)] = jnp.broadcast_to(
                            a, (16,))
                return 0
            lax.fori_loop(0, CHUNK // 16, bg, 0)

        def fire1(k, j, sm):
            s = wid * ta + k * CHUNK
            pltpu.async_copy(dst_h.at[pl.ds(s, CHUNK)], dstv.at[j], sm)
            pltpu.async_copy(ex_h.at[pl.ds(s, CHUNK)],
                             exb.at[pl.ds(j * CHUNK, CHUNK)], sm)

        def finish1(j, sm):
            pltpu.make_async_copy(dst_h.at[pl.ds(0, CHUNK)],
                                  dstv.at[j], sm).wait()
            pltpu.make_async_copy(ex_h.at[pl.ds(0, CHUNK)],
                                  exb.at[pl.ds(j * CHUNK, CHUNK)], sm).wait()
            build1(j)
            pltpu.sync_copy(rows.at[pl.ds(j * CHUNK, CHUNK)],
                            out_sh.at[dstv.at[j]], add=True)

        fire1(0, 0, sem)

        def pair1(k2, _):
            k = k2 * 2
            fire1(k + 1, 1, sem2)
            finish1(0, sem)

            @pl.when(k + 2 < nch_a)
            def _():
                fire1(k + 2, 0, sem)
            finish1(1, sem2)
            return 0
        lax.fori_loop(0, nch_a // 2, pair1, 0)
        if nch_a % 2:
            finish1(0, sem)
        plsc.subcore_barrier()
        pltpu.sync_copy(out_sh.at[pl.ds(sid * stripe, stripe), :],
                        dp_h.at[cid, pl.ds(sid * stripe, stripe), :])
        zero_stripe()
        plsc.subcore_barrier()

        def scale_group2(j, g):
            exv = exb[pl.ds(j * CHUNK + g * 16, 16)]
            for uu in range(16):
                a = exv[uu]
                e = j * CHUNK + g * 16 + uu
                for q in range(dh // 16):
                    sl = pl.ds(q * 16, 16)
                    rows[e, sl] = rows[e, sl] * a

        def fire2(k, j, sm):
            s = sid * tb + k * CHUNK
            pltpu.sync_copy(src_h.at[pl.ds(s, CHUNK)], srcv.at[j])
            pltpu.sync_copy(dst_h.at[pl.ds(s, CHUNK)], dstv.at[j])
            pltpu.sync_copy(ex_h.at[pl.ds(s, CHUNK)],
                            exb.at[pl.ds(j * CHUNK, CHUNK)])
            pltpu.async_copy(xl_h.at[cid].at[srcv.at[j]],
                             rows.at[pl.ds(j * CHUNK, CHUNK)], sm)

        def finish2(j, sm):
            dummy = xl_h.at[0].at[pl.ds(0, CHUNK)]
            pltpu.make_async_copy(
                dummy, rows.at[pl.ds(j * CHUNK, CHUNK)], sm).wait()

            def sg(g, _):
                scale_group2(j, g)
                return 0
            lax.fori_loop(0, CHUNK // 16, sg, 0)
            pltpu.sync_copy(rows.at[pl.ds(j * CHUNK, CHUNK)],
                            out_sh.at[dstv.at[j]], add=True)

        fire2(0, 0, sem)

        def pair2(k2, _):
            k = k2 * 2
            fire2(k + 1, 1, sem2)
            finish2(0, sem)

            @pl.when(k + 2 < nch_b)
            def _():
                fire2(k + 2, 0, sem)
            finish2(1, sem2)
            return 0
        lax.fori_loop(0, nch_b // 2, pair2, 0)
        if nch_b % 2:
            finish2(0, sem)
        plsc.subcore_barrier()
        pltpu.sync_copy(out_sh.at[pl.ds(sid * stripe, stripe), :],
                        out_h.at[cid, pl.ds(sid * stripe, stripe), :])

    kern = pl.kernel(
        body,
        out_type=[jax.ShapeDtypeStruct((NC, npad, dh), jnp.float32),
                  jax.ShapeDtypeStruct((NC, npad, DENW), jnp.float32)],
        mesh=_sc_mesh(),
        scratch_types=[
            pltpu.VMEM((2, CHUNK), jnp.int32),
            pltpu.VMEM((2, CHUNK), jnp.int32),
            pltpu.VMEM((2 * CHUNK, dh), jnp.float32),
            pltpu.VMEM((2 * CHUNK,), jnp.float32),
            pltpu.VMEM((min(16, npad // NS), dh), jnp.float32),
            pltpu.VMEM_SHARED((npad, dh), jnp.float32),
            pltpu.SemaphoreType.DMA,
            pltpu.SemaphoreType.DMA,
        ],
    )
    return kern(xlT, src, dst, ex)


def _phase_l4(xl4, xr4, src, dst, att4):
    npad = xl4.shape[0]
    E = src.shape[0]
    ta, nch = _edge_split(E, NC * NS)
    stripe = npad // NS
    zbr = min(16, stripe)

    def body(xl_h, xr_h, src_h, dst_h, att_h, u_h, dp_h,
             srcv, dstv, xsg, xrg, rows, attv, zb, acc_sh, sem):
        cid = lax.axis_index("c")
        sid = lax.axis_index("s")
        wid = sid * NC + cid
        base = wid * ta
        zero16 = jnp.zeros((16,), jnp.float32)

        def zero_stripe():
            for t in range(stripe // zbr):
                pltpu.sync_copy(
                    zb, acc_sh.at[pl.ds(sid * stripe + t * zbr, zbr), :])

        for r in range(zbr):
            for q in range(DENW // 16):
                zb[r, pl.ds(q * 16, 16)] = zero16
        zero_stripe()
        pltpu.sync_copy(att_h, attv)
        plsc.subcore_barrier()
        a0 = attv[pl.ds(0, 16)][0]

        def pass_edges(numerator):
            def chunk(k, _):
                s = base + k * CHUNK
                pltpu.sync_copy(src_h.at[pl.ds(s, CHUNK)], srcv.at[0])
                pltpu.sync_copy(dst_h.at[pl.ds(s, CHUNK)], dstv.at[0])
                cps = [pltpu.async_copy(xl_h.at[srcv.at[0]], xsg, sem),
                       pltpu.async_copy(xr_h.at[dstv.at[0]], xrg, sem)]
                for cp in cps:
                    cp.wait()

                def bg(g, _):
                    sl = pl.ds(g * 16, 16)
                    l = xsg[sl]
                    t = l + xrg[sl]
                    t = jnp.maximum(t, 0.2 * t)
                    v = jnp.exp(a0 * t)
                    if numerator:
                        v = v * l
                    for uu in range(16):
                        e = g * 16 + uu
                        for q in range(DENW // 16):
                            rows[e, pl.ds(q * 16, 16)] = jnp.broadcast_to(
                                v[uu], (16,))
                    return 0
                lax.fori_loop(0, CHUNK // 16, bg, 0)
                pltpu.sync_copy(rows, acc_sh.at[dstv.at[0]], add=True)
                return 0
            lax.fori_loop(0, nch, chunk, 0)

        pass_edges(False)
        plsc.subcore_barrier()
        off = pl.ds(sid * stripe, stripe)
        pltpu.sync_copy(acc_sh.at[off, :], dp_h.at[cid, off, :])
        zero_stripe()
        plsc.subcore_barrier()
        pass_edges(True)
        plsc.subcore_barrier()
        pltpu.sync_copy(acc_sh.at[off, :], u_h.at[cid, off, :])

    kern = pl.kernel(
        body,
        out_type=[jax.ShapeDtypeStruct((NC, npad, DENW), jnp.float32),
                  jax.ShapeDtypeStruct((NC, npad, DENW), jnp.float32)],
        mesh=_sc_mesh(),
        scratch_types=[
            pltpu.VMEM((2, CHUNK), jnp.int32),
            pltpu.VMEM((2, CHUNK), jnp.int32),
            pltpu.VMEM((CHUNK,), jnp.float32),
            pltpu.VMEM((CHUNK,), jnp.float32),
            pltpu.VMEM((CHUNK, DENW), jnp.float32),
            pltpu.VMEM((16,), jnp.float32),
            pltpu.VMEM((min(16, npad // NS), DENW), jnp.float32),
            pltpu.VMEM_SHARED((npad, DENW), jnp.float32),
            pltpu.SemaphoreType.DMA,
        ],
    )
    return kern(xl4, xr4, src, dst, att4)


def _gat_layer(h_or_x, b_prev, dnorm, p, src, dst, halves_in, relu_in):
    xlT, xrT = _mm(h_or_x, b_prev, dnorm, p["Wl"], p["bl"], p["Wr"], p["br"],
                   halves_in, relu_in)
    ex = _phase_a(xlT, xrT, src, dst, p["att"])
    u, dp = _phase_b(xlT, src, dst, ex)
    return u, dp


def kernel(x, params, edge_index):
    n, _ = x.shape
    npad = ((n + 16 + 255) // 256) * 256
    xp = jnp.pad(x, ((0, npad - n), (0, 0)))
    e = edge_index.shape[1]
    epad = -(-e // (CHUNK * NC * NS)) * (CHUNK * NC * NS)
    dump = npad - 16
    src = jnp.pad(edge_index[0], (0, epad - e))
    dst = jnp.pad(edge_index[1], (0, epad - e), constant_values=dump)
    p1, p2, p3, p4 = (params["l1"], params["l2"], params["l3"], params["l4"])

    u, dp = _gat_layer(xp, None, None, p1, src, dst,
                       halves_in=False, relu_in=False)
    u, dp = _gat_layer(u, p1["bias"], dp, p2, src, dst,
                       halves_in=True, relu_in=True)
    u, dp = _gat_layer(u, p2["bias"], dp, p3, src, dst,
                       halves_in=True, relu_in=True)

    pad = 128
    wl4 = jnp.pad(p4["Wl"], ((0, 0), (0, pad - 1)))
    wr4 = jnp.pad(p4["Wr"], ((0, 0), (0, pad - 1)))
    bl4 = jnp.pad(p4["bl"], (0, pad - 1))
    br4 = jnp.pad(p4["br"], (0, pad - 1))
    xlT4, xrT4 = _mm(u, p3["bias"], dp, wl4, bl4, wr4, br4,
                     halves_in=True, relu_in=True)
    xl4 = xlT4[0, :, 0]
    xr4 = xrT4[0, :, 0]
    att4 = jnp.pad(p4["att"], (0, 15))
    u4, dp4 = _phase_l4(xl4, xr4, src, dst, att4)
    o = _final_norm(u4, dp4, params["l4"]["bias"][0])
    return o[:n, 0]
